# baseline (speedup 1.0000x reference)
import jax
import jax.numpy as jnp
from jax import lax
from jax.experimental import pallas as pl
from jax.experimental.pallas import tpu as pltpu
from jax.experimental.pallas import tpu_sc as plsc

N = 8388608
NC = 2
NS = 16
NW = NC * NS
LANES = 16
TBL = 128
CHUNK = 8192
NBUF = 4
UNROLL = 8

SC_N = 14 * NW * CHUNK
TC_N = N - SC_N
PER_W = SC_N // NW
NCHUNK = PER_W // CHUNK

LN = 128
TC_ROWS = TC_N // LN
BR = 4096


def _sc_body(e_hbm, z_hbm, tbl_hbm, out_hbm, tbl_v, *bufs):
    z_bufs = bufs[0:NBUF]
    e_bufs = bufs[NBUF:2 * NBUF]
    sem_in = bufs[2 * NBUF]
    sem_out = bufs[2 * NBUF + 1]

    wid = lax.axis_index("s") * NC + lax.axis_index("c")
    start = wid * PER_W

    pltpu.sync_copy(tbl_hbm, tbl_v)

    in_handles = [None] * NCHUNK
    out_handles = [None] * NCHUNK

    def start_in(g):
        b = g % NBUF
        base = start + g * CHUNK
        h_e = pltpu.async_copy(e_hbm.at[pl.ds(base, CHUNK)], e_bufs[b],
                               sem_in.at[b])
        h_z = pltpu.async_copy(z_hbm.at[pl.ds(base, CHUNK)], z_bufs[b],
                               sem_in.at[b])
        in_handles[g] = (h_e, h_z)

    for g in range(min(2, NCHUNK)):
        start_in(g)

    for g in range(NCHUNK):
        b = g % NBUF
        if g + 2 < NCHUNK:
            if g - 2 >= 0:
                out_handles[g - 2].wait()
            start_in(g + 2)
        h_e, h_z = in_handles[g]
        h_e.wait()
        h_z.wait()

        z_v = z_bufs[b]
        e_v = e_bufs[b]

        @plsc.parallel_loop(0, CHUNK, step=LANES, unroll=UNROLL)
        def _(i):
            idx = z_v[pl.ds(i, LANES)]
            e = e_v[pl.ds(i, LANES)]
            w = plsc.load_gather(tbl_v, [idx])
            sc = plsc.bitcast(w & jnp.int32(-65536), jnp.float32)
            sh = plsc.bitcast(w << 16, jnp.float32)
            e_v[pl.ds(i, LANES)] = sc * e + sh

        base = start + g * CHUNK
        out_handles[g] = pltpu.async_copy(
            e_v, out_hbm.at[pl.ds(base, CHUNK)], sem_out.at[b])

    for g in range(max(0, NCHUNK - 2), NCHUNK):
        out_handles[g].wait()


def _tc_body(tbl_ref, e_ref, z_ref, o_ref):
    z = z_ref[...]
    e = e_ref[...]
    t = jnp.broadcast_to(tbl_ref[...].reshape((1, LN)), z.shape)
    w = jnp.take_along_axis(t, z, axis=-1)
    sc = lax.bitcast_convert_type(w & jnp.int32(-65536), jnp.float32)
    sh = lax.bitcast_convert_type(w << 16, jnp.float32)
    o_ref[...] = sc * e + sh


def kernel(atomic_energies, atomic_numbers, scale, shift):
    e = atomic_energies.astype(jnp.float32)
    z = atomic_numbers.astype(jnp.int32)
    pad = TBL - scale.shape[0]
    sc16 = lax.bitcast_convert_type(
        scale.astype(jnp.bfloat16), jnp.uint16).astype(jnp.uint32)
    sh16 = lax.bitcast_convert_type(
        shift.astype(jnp.bfloat16), jnp.uint16).astype(jnp.uint32)
    tbl = ((sc16 << 16) | sh16).astype(jnp.int32)
    tbl_p = jnp.pad(tbl, (0, pad))

    sc_run = pl.kernel(
        _sc_body,
        mesh=plsc.VectorSubcoreMesh(core_axis_name="c", subcore_axis_name="s"),
        out_type=jax.ShapeDtypeStruct((SC_N,), jnp.float32),
        compiler_params=pltpu.CompilerParams(needs_layout_passes=False),
        scratch_types=(
            [pltpu.VMEM((TBL,), jnp.int32)]
            + [pltpu.VMEM((CHUNK,), jnp.int32) for _ in range(NBUF)]
            + [pltpu.VMEM((CHUNK,), jnp.float32) for _ in range(NBUF)]
            + [pltpu.SemaphoreType.DMA((NBUF,)),
               pltpu.SemaphoreType.DMA((NBUF,))]
        ),
    )
    sc_out = sc_run(e[:SC_N], z[:SC_N], tbl_p)

    tc_out = pl.pallas_call(
        _tc_body,
        grid=(TC_ROWS // BR,),
        in_specs=[
            pl.BlockSpec((1, TBL), lambda i: (0, 0)),
            pl.BlockSpec((BR, LN), lambda i: (i, 0)),
            pl.BlockSpec((BR, LN), lambda i: (i, 0)),
        ],
        out_specs=pl.BlockSpec((BR, LN), lambda i: (i, 0)),
        out_shape=jax.ShapeDtypeStruct((TC_ROWS, LN), jnp.float32),
    )(tbl_p.reshape(1, TBL), e[SC_N:].reshape(TC_ROWS, LN),
      z[SC_N:].reshape(TC_ROWS, LN))

    return jnp.concatenate([sc_out, tc_out.reshape(TC_N)])


# speedup vs baseline: 1.8932x; 1.8932x over previous
import jax
import jax.numpy as jnp
from jax import lax
from jax.experimental import pallas as pl
from jax.experimental.pallas import tpu as pltpu
from jax.experimental.pallas import tpu_sc as plsc

N = 8388608
NC = 2
NS = 16
NW = NC * NS
LANES = 16
TBL = 128
CHUNK = 8192
NBUF = 4
UNROLL = 8

SC_N = 14 * NW * CHUNK
TC_N = N - SC_N
PER_W = SC_N // NW
NCHUNK = PER_W // CHUNK

LN = 128
ROWS = N // LN
SC_ROWS = SC_N // LN
BR = 4096
SC_BLKS = SC_ROWS // BR
TC_BLKS = (ROWS - SC_ROWS) // BR


def _sc_body(e_hbm, z_hbm, tbl_hbm, out_hbm, tbl_v, *bufs):
    z_bufs = bufs[0:NBUF]
    e_bufs = bufs[NBUF:2 * NBUF]
    sem_in = bufs[2 * NBUF]
    sem_out = bufs[2 * NBUF + 1]

    wid = lax.axis_index("s") * NC + lax.axis_index("c")
    start = wid * PER_W

    pltpu.sync_copy(tbl_hbm, tbl_v)

    in_handles = [None] * NCHUNK
    out_handles = [None] * NCHUNK

    def start_in(g):
        b = g % NBUF
        base = start + g * CHUNK
        h_e = pltpu.async_copy(e_hbm.at[pl.ds(base, CHUNK)], e_bufs[b],
                               sem_in.at[b])
        h_z = pltpu.async_copy(z_hbm.at[pl.ds(base, CHUNK)], z_bufs[b],
                               sem_in.at[b])
        in_handles[g] = (h_e, h_z)

    for g in range(min(2, NCHUNK)):
        start_in(g)

    for g in range(NCHUNK):
        b = g % NBUF
        if g + 2 < NCHUNK:
            if g - 2 >= 0:
                out_handles[g - 2].wait()
            start_in(g + 2)
        h_e, h_z = in_handles[g]
        h_e.wait()
        h_z.wait()

        z_v = z_bufs[b]
        e_v = e_bufs[b]

        @plsc.parallel_loop(0, CHUNK, step=LANES, unroll=UNROLL)
        def _(i):
            idx = z_v[pl.ds(i, LANES)]
            e = e_v[pl.ds(i, LANES)]
            w = plsc.load_gather(tbl_v, [idx])
            sc = plsc.bitcast(w & jnp.int32(-65536), jnp.float32)
            sh = plsc.bitcast(w << 16, jnp.float32)
            e_v[pl.ds(i, LANES)] = sc * e + sh

        base = start + g * CHUNK
        out_handles[g] = pltpu.async_copy(
            e_v, out_hbm.at[pl.ds(base, CHUNK)], sem_out.at[b])

    for g in range(max(0, NCHUNK - 2), NCHUNK):
        out_handles[g].wait()


def _tc_body(buf_ref, tbl_ref, e_ref, z_ref, o_ref):
    del buf_ref
    z = z_ref[...]
    e = e_ref[...]
    t = jnp.broadcast_to(tbl_ref[...].reshape((1, LN)), z.shape)
    w = jnp.take_along_axis(t, z, axis=-1)
    sc = lax.bitcast_convert_type(w & jnp.int32(-65536), jnp.float32)
    sh = lax.bitcast_convert_type(w << 16, jnp.float32)
    o_ref[...] = sc * e + sh


def kernel(atomic_energies, atomic_numbers, scale, shift):
    e = atomic_energies.astype(jnp.float32)
    z = atomic_numbers.astype(jnp.int32)
    pad = TBL - scale.shape[0]
    sc16 = lax.bitcast_convert_type(
        scale.astype(jnp.bfloat16), jnp.uint16).astype(jnp.uint32)
    sh16 = lax.bitcast_convert_type(
        shift.astype(jnp.bfloat16), jnp.uint16).astype(jnp.uint32)
    tbl = ((sc16 << 16) | sh16).astype(jnp.int32)
    tbl_p = jnp.pad(tbl, (0, pad))

    sc_run = pl.kernel(
        _sc_body,
        mesh=plsc.VectorSubcoreMesh(core_axis_name="c", subcore_axis_name="s"),
        out_type=jax.ShapeDtypeStruct((N,), jnp.float32),
        compiler_params=pltpu.CompilerParams(needs_layout_passes=False),
        scratch_types=(
            [pltpu.VMEM((TBL,), jnp.int32)]
            + [pltpu.VMEM((CHUNK,), jnp.int32) for _ in range(NBUF)]
            + [pltpu.VMEM((CHUNK,), jnp.float32) for _ in range(NBUF)]
            + [pltpu.SemaphoreType.DMA((NBUF,)),
               pltpu.SemaphoreType.DMA((NBUF,))]
        ),
    )
    sc_out = sc_run(e, z, tbl_p)

    out = pl.pallas_call(
        _tc_body,
        grid=(TC_BLKS,),
        in_specs=[
            pl.BlockSpec(memory_space=pltpu.MemorySpace.HBM),
            pl.BlockSpec((1, TBL), lambda i: (0, 0)),
            pl.BlockSpec((BR, LN), lambda i: (SC_BLKS + i, 0)),
            pl.BlockSpec((BR, LN), lambda i: (SC_BLKS + i, 0)),
        ],
        out_specs=pl.BlockSpec((BR, LN), lambda i: (SC_BLKS + i, 0)),
        out_shape=jax.ShapeDtypeStruct((ROWS, LN), jnp.float32),
        input_output_aliases={0: 0},
    )(sc_out.reshape(ROWS, LN), tbl_p.reshape(1, TBL),
      e.reshape(ROWS, LN), z.reshape(ROWS, LN))

    return out.reshape(N)
